# Initial kernel scaffold; baseline (speedup 1.0000x reference)
import jax
import jax.numpy as jnp
from jax.experimental import pallas as pl


def kernel(h, edge_index, edge_id, W, b, relvectors):
    raise NotImplementedError("write your pallas kernel here")


# speedup vs baseline: 6.4429x; 6.4429x over previous
import functools

import jax
import jax.numpy as jnp
from jax import lax
from jax.experimental import pallas as pl
from jax.experimental.pallas import tpu as pltpu
from jax.experimental.pallas import tpu_sc as plsc

NC = 2
NS = 16
NW = NC * NS
L = 16
CHUNK = 128


def _build_table(n_rels, h_ref, rel_ref, out_ref):
    hv = h_ref[...]
    for r in range(n_rels):
        out_ref[r] = jnp.maximum(hv + rel_ref[r], 0.0)


def _build_gidx(n_nodes, src_ref, eid_ref, out_ref):
    out_ref[...] = eid_ref[...] * n_nodes + src_ref[...]


def _finish(ps_ref, pc_ref, w_ref, b_ref, o_ref):
    s = ps_ref[0] + ps_ref[1]
    c = pc_ref[0][:, :1] + pc_ref[1][:, :1]
    red = s / jnp.maximum(c, 1.0)
    o_ref[...] = (
        lax.dot_general(red, w_ref[...], (((1,), (1,)), ((), ())),
                        preferred_element_type=jnp.float32)
        + b_ref[...]
    )


def _make_sc_scatter(n_nodes, dim, n_rels, cpw, acc_rows):
    rps = acc_rows // NS
    zr = rps // 2
    mesh = plsc.VectorSubcoreMesh(core_axis_name="c", subcore_axis_name="s")

    @functools.partial(
        pl.kernel,
        mesh=mesh,
        compiler_params=pltpu.CompilerParams(use_tc_tiling_on_sc=False),
        out_type=[
            jax.ShapeDtypeStruct((NC, acc_rows, dim), jnp.float32),
            jax.ShapeDtypeStruct((NC, acc_rows, L), jnp.float32),
        ],
        scratch_types=[
            pltpu.VMEM((CHUNK,), jnp.int32),
            pltpu.VMEM((CHUNK,), jnp.int32),
            pltpu.VMEM((CHUNK, dim), jnp.float32),
            pltpu.VMEM((CHUNK, L), jnp.float32),
            pltpu.VMEM((rps, L), jnp.float32),
            pltpu.VMEM_SHARED((acc_rows, dim), jnp.float32),
            pltpu.VMEM_SHARED((acc_rows, L), jnp.float32),
            pltpu.SemaphoreType.DMA,
        ],
    )
    def sc_kernel(r_hbm, gidx_hbm, dst_hbm, psum_hbm, pcnt_hbm,
                  gidx_c, dst_c, rows_v, ones_v, zcnt_v,
                  acc_s, cnt_s, sem):
        cid = lax.axis_index("c")
        sid = lax.axis_index("s")
        wid = sid * NC + cid

        zeros16 = jnp.zeros((L,), jnp.float32)
        ones16 = jnp.ones((L,), jnp.float32)

        def zr_body(k, _):
            rows_v[k // (dim // L), pl.ds((k % (dim // L)) * L, L)] = zeros16
            return 0
        lax.fori_loop(0, CHUNK * (dim // L), zr_body, 0)

        def zc_body(k, _):
            zcnt_v[k, :] = zeros16
            return 0
        lax.fori_loop(0, rps, zc_body, 0)

        def on_body(k, _):
            ones_v[k, :] = ones16
            return 0
        lax.fori_loop(0, CHUNK, on_body, 0)

        base = sid * rps
        nfull, rem = rps // CHUNK, rps % CHUNK
        for k in range(nfull):
            pltpu.sync_copy(rows_v, acc_s.at[pl.ds(base + k * CHUNK, CHUNK)])
        if rem:
            pltpu.sync_copy(rows_v.at[pl.ds(0, rem)],
                            acc_s.at[pl.ds(base + nfull * CHUNK, rem)])
        pltpu.sync_copy(zcnt_v, cnt_s.at[pl.ds(base, rps)])
        plsc.subcore_barrier()

        def chunk_body(c, _):
            pltpu.sync_copy(gidx_hbm.at[wid, c], gidx_c)
            pltpu.sync_copy(dst_hbm.at[wid, c], dst_c)
            pltpu.async_copy(r_hbm.at[gidx_c], rows_v, sem).wait()
            pltpu.sync_copy(rows_v, acc_s.at[dst_c], add=True)
            pltpu.sync_copy(ones_v, cnt_s.at[dst_c], add=True)
            return 0
        lax.fori_loop(0, cpw, chunk_body, 0)

        plsc.subcore_barrier()
        pltpu.sync_copy(acc_s.at[pl.ds(base, rps)],
                        psum_hbm.at[cid, pl.ds(base, rps)])
        pltpu.sync_copy(cnt_s.at[pl.ds(base, rps)],
                        pcnt_hbm.at[cid, pl.ds(base, rps)])

    return sc_kernel


def kernel(h, edge_index, edge_id, W, b, relvectors):
    n_nodes, dim = h.shape
    n_rels = relvectors.shape[0]
    n_edges = edge_index.shape[1]

    src = edge_index[0].astype(jnp.int32)
    dst = edge_index[1].astype(jnp.int32)
    eid = edge_id.astype(jnp.int32)

    cpw = -(-n_edges // (NW * CHUNK))
    epad = NW * cpw * CHUNK
    pad = epad - n_edges
    if pad:
        src = jnp.concatenate([src, jnp.zeros((pad,), jnp.int32)])
        eid = jnp.concatenate([eid, jnp.zeros((pad,), jnp.int32)])
        dst = jnp.concatenate([dst, jnp.full((pad,), n_nodes, jnp.int32)])

    acc_rows = -(-(n_nodes + (1 if pad else 0)) // (8 * NS)) * (8 * NS)

    nbs = 1000
    table = pl.pallas_call(
        functools.partial(_build_table, n_rels),
        grid=(n_nodes // nbs,),
        in_specs=[
            pl.BlockSpec((nbs, dim), lambda i: (i, 0)),
            pl.BlockSpec((n_rels, dim), lambda i: (0, 0)),
        ],
        out_specs=pl.BlockSpec((n_rels, nbs, dim), lambda i: (0, i, 0)),
        out_shape=jax.ShapeDtypeStruct((n_rels, n_nodes, dim), jnp.float32),
    )(h, relvectors).reshape(n_rels * n_nodes, dim)

    src2 = src.reshape(cpw, NW * CHUNK)
    eid2 = eid.reshape(cpw, NW * CHUNK)
    gidx = pl.pallas_call(
        functools.partial(_build_gidx, n_nodes),
        out_shape=jax.ShapeDtypeStruct((cpw, NW * CHUNK), jnp.int32),
    )(src2, eid2)

    gidx3 = gidx.reshape(NW, cpw, CHUNK)
    dst3 = dst.reshape(NW, cpw, CHUNK)

    psum, pcnt = _make_sc_scatter(n_nodes, dim, n_rels, cpw, acc_rows)(
        table, gidx3, dst3)

    rbs = 1000
    out = pl.pallas_call(
        _finish,
        grid=(n_nodes // rbs,),
        in_specs=[
            pl.BlockSpec((NC, rbs, dim), lambda i: (0, i, 0)),
            pl.BlockSpec((NC, rbs, L), lambda i: (0, i, 0)),
            pl.BlockSpec((dim, dim), lambda i: (0, 0)),
            pl.BlockSpec((1, dim), lambda i: (0, 0)),
        ],
        out_specs=pl.BlockSpec((rbs, dim), lambda i: (i, 0)),
        out_shape=jax.ShapeDtypeStruct((n_nodes, dim), jnp.float32),
    )(psum, pcnt, W, b.reshape(1, dim))
    return out
